# baseline (speedup 1.0000x reference)
import jax
import jax.numpy as jnp
from jax.experimental import pallas as pl

_N = 8192
_BR = 256


def _body(x_ref, o_ref):
    sub = x_ref[...]
    rows = jax.lax.broadcasted_iota(jnp.int32, (_BR, _BR), 0)
    cols = jax.lax.broadcasted_iota(jnp.int32, (_BR, _BR), 1)
    o_ref[...] = jnp.where(rows == cols, jnp.tanh(sub), sub)


def kernel(x):
    n = x.shape[0]
    return pl.pallas_call(
        _body,
        grid=(n // _BR,),
        in_specs=[pl.BlockSpec((_BR, _BR), lambda i: (i, i))],
        out_specs=pl.BlockSpec((_BR, _BR), lambda i: (i, i)),
        out_shape=jax.ShapeDtypeStruct((n, n), x.dtype),
        input_output_aliases={0: 0},
    )(x)
